# baseline (speedup 1.0000x reference)
import jax
import jax.numpy as jnp
from jax import lax
from jax.experimental import pallas as pl
from jax.experimental.pallas import tpu as pltpu
from jax.experimental.pallas import tpu_sc as plsc

B = 16384
H = 50
D = 32
NC, NS = 2, 16
NW = NC * NS
BPW = B // NW
CB = 16
NCH = BPW // CB
HALF = D // 2
HP = 56


def _sc_body(users_h, seqs_h, pos_h, neg_h, uw_h, iw_h, out_h,
             score_p, score_n,
             s_idx_a, s_idx_b, s_rows_a, s_rows_b,
             u_idx_a, u_idx_b, p_idx_a, p_idx_b, n_idx_a, n_idx_b,
             u_rows_a, u_rows_b, p_rows_a, p_rows_b, n_rows_a, n_rows_b,
             sem_a, sem_b):
    wid = lax.axis_index("s") * NC + lax.axis_index("c")
    base_w = wid * BPW
    lane = lax.iota(jnp.int32, HALF)

    bufs = ((s_idx_a, s_rows_a, u_idx_a, u_rows_a, p_idx_a, p_rows_a,
             n_idx_a, n_rows_a, sem_a),
            (s_idx_b, s_rows_b, u_idx_b, u_rows_b, p_idx_b, p_rows_b,
             n_idx_b, n_rows_b, sem_b))

    def fire(c, buf):
        s_idx, s_rows, u_idx, u_rows, p_idx, p_rows, n_idx, n_rows, sem = buf
        cbase = base_w + c * CB
        pltpu.sync_copy(seqs_h.at[pl.ds(cbase * HP, CB * HP)], s_idx)
        pltpu.sync_copy(users_h.at[pl.ds(cbase, CB)], u_idx)
        pltpu.sync_copy(pos_h.at[pl.ds(cbase, CB)], p_idx)
        pltpu.sync_copy(neg_h.at[pl.ds(cbase, CB)], n_idx)
        pltpu.async_copy(uw_h.at[u_idx], u_rows, sem)
        pltpu.async_copy(iw_h.at[p_idx], p_rows, sem)
        pltpu.async_copy(iw_h.at[n_idx], n_rows, sem)
        for e in range(CB):
            pltpu.async_copy(iw_h.at[s_idx.at[pl.ds(e * HP, H)]],
                             s_rows.at[pl.ds(e * H, H), :], sem)

    def drain(buf):
        s_idx, s_rows, u_idx, u_rows, p_idx, p_rows, n_idx, n_rows, sem = buf
        pltpu.make_async_copy(uw_h.at[u_idx], u_rows, sem).wait()
        pltpu.make_async_copy(iw_h.at[p_idx], p_rows, sem).wait()
        pltpu.make_async_copy(iw_h.at[n_idx], n_rows, sem).wait()
        for e in range(CB):
            pltpu.make_async_copy(iw_h.at[s_idx.at[pl.ds(e * HP, H)]],
                                  s_rows.at[pl.ds(e * H, H), :], sem).wait()

    def compute(c, buf):
        s_idx, s_rows, u_idx, u_rows, p_idx, p_rows, n_idx, n_rows, sem = buf

        def elem_body(l, carry):
            pos_vec, neg_vec = carry
            eb = l * H
            acc0 = s_rows[eb, pl.ds(0, HALF)]
            acc1 = s_rows[eb, pl.ds(HALF, HALF)]
            for j in range(1, H):
                acc0 = acc0 + s_rows[eb + j, pl.ds(0, HALF)]
                acc1 = acc1 + s_rows[eb + j, pl.ds(HALF, HALF)]
            f0 = u_rows[l, pl.ds(0, HALF)] + acc0 * (1.0 / H)
            f1 = u_rows[l, pl.ds(HALF, HALF)] + acc1 * (1.0 / H)
            ps = jnp.sum(f0 * p_rows[l, pl.ds(0, HALF)]
                         + f1 * p_rows[l, pl.ds(HALF, HALF)])
            ns = jnp.sum(f0 * n_rows[l, pl.ds(0, HALF)]
                         + f1 * n_rows[l, pl.ds(HALF, HALF)])
            pos_vec = jnp.where(lane == l, ps, pos_vec)
            neg_vec = jnp.where(lane == l, ns, neg_vec)
            return pos_vec, neg_vec

        z = jnp.zeros((HALF,), jnp.float32)
        pos_vec, neg_vec = lax.fori_loop(0, CB, elem_body, (z, z))
        score_p[pl.ds(c * CB, CB)] = pos_vec
        score_n[pl.ds(c * CB, CB)] = neg_vec

    fire(0, bufs[0])

    def pair_body(cp, _):
        for p in (0, 1):
            c = cp * 2 + p
            cn = lax.rem(c + 1, NCH)
            fire(cn, bufs[1 - p])
            drain(bufs[p])
            compute(c, bufs[p])
        return 0

    lax.fori_loop(0, NCH // 2, pair_body, 0)
    drain(bufs[0])

    pltpu.sync_copy(score_p, out_h.at[0, pl.ds(base_w, BPW)])
    pltpu.sync_copy(score_n, out_h.at[1, pl.ds(base_w, BPW)])


@jax.jit
def _run(users, seqs, posItems, negItems, emb_user_w, emb_item_w):
    mesh = plsc.VectorSubcoreMesh(core_axis_name="c", subcore_axis_name="s",
                                  num_cores=NC, num_subcores=NS)
    f = pl.kernel(
        _sc_body,
        out_type=jax.ShapeDtypeStruct((2, B), jnp.float32),
        mesh=mesh,
        scratch_types=[
            pltpu.VMEM((BPW,), jnp.float32),
            pltpu.VMEM((BPW,), jnp.float32),
            pltpu.VMEM((CB * HP,), jnp.int32),
            pltpu.VMEM((CB * HP,), jnp.int32),
            pltpu.VMEM((CB * H, D), jnp.float32),
            pltpu.VMEM((CB * H, D), jnp.float32),
            pltpu.VMEM((CB,), jnp.int32),
            pltpu.VMEM((CB,), jnp.int32),
            pltpu.VMEM((CB,), jnp.int32),
            pltpu.VMEM((CB,), jnp.int32),
            pltpu.VMEM((CB,), jnp.int32),
            pltpu.VMEM((CB,), jnp.int32),
            pltpu.VMEM((CB, D), jnp.float32),
            pltpu.VMEM((CB, D), jnp.float32),
            pltpu.VMEM((CB, D), jnp.float32),
            pltpu.VMEM((CB, D), jnp.float32),
            pltpu.VMEM((CB, D), jnp.float32),
            pltpu.VMEM((CB, D), jnp.float32),
            pltpu.SemaphoreType.DMA,
            pltpu.SemaphoreType.DMA,
        ],
        compiler_params=pltpu.CompilerParams(use_tc_tiling_on_sc=False,
                                             needs_layout_passes=False),
    )
    return f(users, seqs, posItems, negItems, emb_user_w, emb_item_w)


def kernel(users, seqs, posItems, negItems, emb_user_w, emb_item_w):
    seqs_flat = jnp.pad(seqs, ((0, 0), (0, HP - H))).reshape(-1)
    return _run(users, seqs_flat, posItems, negItems,
                emb_user_w, emb_item_w)


# speedup vs baseline: 1.1224x; 1.1224x over previous
import jax
import jax.numpy as jnp
from jax import lax
from jax.experimental import pallas as pl
from jax.experimental.pallas import tpu as pltpu
from jax.experimental.pallas import tpu_sc as plsc

B = 16384
H = 50
D = 32
NC, NS = 2, 16
NW = NC * NS
BPW = B // NW
CB = 16
NCH = BPW // CB
HALF = D // 2
HP = 56


UNIT = 512
NU = V_ROWS = 1000000
NUF = V_ROWS // UNIT
TAIL = V_ROWS - NUF * UNIT
KMAX = (NUF + NW - 1) // NW
MPITCH = D + 1


def _tp_body(utv_h, ut16_h, uout_h, win, mid, obuf):
    wid = lax.axis_index("s") * NC + lax.axis_index("c")
    lane = lax.iota(jnp.int32, HALF)

    def do_unit(col0, r0):
        pltpu.sync_copy(utv_h.at[:, pl.ds(col0, UNIT)], win)

        def d_body(d, _):
            for k2 in range(UNIT // HALF):
                vals = win[d, pl.ds(k2 * HALF, HALF)]
                idx = (lane + k2 * HALF) * MPITCH + d
                plsc.store_scatter(mid, [idx], vals)
            return 0

        lax.fori_loop(0, D, d_body, 0)

        def r_body(r, _):
            rb = r * (4 * MPITCH)
            for a in range(4):
                ob = a * D
                obuf[r, pl.ds(ob, HALF)] = mid[pl.ds(rb + a * MPITCH, HALF)]
                obuf[r, pl.ds(ob + HALF, HALF)] = (
                    mid[pl.ds(rb + a * MPITCH + HALF, HALF)])
            return 0

        lax.fori_loop(0, UNIT // 4, r_body, 0)
        pltpu.sync_copy(obuf, uout_h.at[pl.ds(r0, UNIT * D // 128), :])

    def unit_body(k, _):
        u = wid + k * NW

        @pl.when(u < NUF)
        def _():
            do_unit(u * UNIT, u * (UNIT * D // 128))
        return 0

    lax.fori_loop(0, KMAX, unit_body, 0)

    @pl.when(wid == 0)
    def _():
        tr = TAIL * D // 128
        r0 = NUF * (UNIT * D // 128)
        pltpu.sync_copy(ut16_h, obuf.at[pl.ds(0, tr), :])
        pltpu.sync_copy(obuf.at[pl.ds(0, tr), :], uout_h.at[pl.ds(r0, tr), :])


def _transpose_user(utv, ut16):
    mesh = plsc.VectorSubcoreMesh(core_axis_name="c", subcore_axis_name="s",
                                  num_cores=NC, num_subcores=NS)
    f = pl.kernel(
        _tp_body,
        out_type=jax.ShapeDtypeStruct((V_ROWS // 4, 128), jnp.float32),
        mesh=mesh,
        scratch_types=[
            pltpu.VMEM((D, UNIT), jnp.float32),
            pltpu.VMEM((UNIT * MPITCH,), jnp.float32),
            pltpu.VMEM((UNIT * D // 128, 128), jnp.float32),
        ],
        compiler_params=pltpu.CompilerParams(use_tc_tiling_on_sc=True,
                                             needs_layout_passes=False,
                                             disable_bounds_checks=True),
    )
    return f(utv, ut16)


def _sc_body(users_h, seqs_h, pos_h, neg_h, uw_h, iw_h, out_h,
             score_p, score_n,
             s_idx_a, s_idx_b, s_rows_a, s_rows_b,
             u_idx_a, u_idx_b, p_idx_a, p_idx_b, n_idx_a, n_idx_b,
             u_rows_a, u_rows_b, p_rows_a, p_rows_b, n_rows_a, n_rows_b,
             sem_a, sem_b):
    wid = lax.axis_index("s") * NC + lax.axis_index("c")
    base_w = wid * BPW
    lane = lax.iota(jnp.int32, HALF)

    bufs = ((s_idx_a, s_rows_a, u_idx_a, u_rows_a, p_idx_a, p_rows_a,
             n_idx_a, n_rows_a, sem_a),
            (s_idx_b, s_rows_b, u_idx_b, u_rows_b, p_idx_b, p_rows_b,
             n_idx_b, n_rows_b, sem_b))

    def fire(c, buf):
        s_idx, s_rows, u_idx, u_rows, p_idx, p_rows, n_idx, n_rows, sem = buf
        cbase = base_w + c * CB
        pltpu.sync_copy(seqs_h.at[pl.ds(cbase * HP, CB * HP)], s_idx)
        pltpu.sync_copy(users_h.at[pl.ds(cbase, CB)], u_idx)
        pltpu.sync_copy(pos_h.at[pl.ds(cbase, CB)], p_idx)
        pltpu.sync_copy(neg_h.at[pl.ds(cbase, CB)], n_idx)
        pltpu.async_copy(uw_h.at[u_idx], u_rows, sem)
        pltpu.async_copy(iw_h.at[p_idx], p_rows, sem)
        pltpu.async_copy(iw_h.at[n_idx], n_rows, sem)
        for e in range(CB):
            pltpu.async_copy(iw_h.at[s_idx.at[pl.ds(e * HP, H)]],
                             s_rows.at[pl.ds(e * H, H), :], sem)

    def drain(buf):
        s_idx, s_rows, u_idx, u_rows, p_idx, p_rows, n_idx, n_rows, sem = buf
        pltpu.make_async_copy(uw_h.at[u_idx], u_rows, sem).wait()
        pltpu.make_async_copy(iw_h.at[p_idx], p_rows, sem).wait()
        pltpu.make_async_copy(iw_h.at[n_idx], n_rows, sem).wait()
        for e in range(CB):
            pltpu.make_async_copy(iw_h.at[s_idx.at[pl.ds(e * HP, H)]],
                                  s_rows.at[pl.ds(e * H, H), :], sem).wait()

    def compute(c, buf):
        s_idx, s_rows, u_idx, u_rows, p_idx, p_rows, n_idx, n_rows, sem = buf

        def elem_body(l, carry):
            pos_vec, neg_vec = carry
            eb = l * H
            acc0 = s_rows[eb, pl.ds(0, HALF)]
            acc1 = s_rows[eb, pl.ds(HALF, HALF)]
            for j in range(1, H):
                acc0 = acc0 + s_rows[eb + j, pl.ds(0, HALF)]
                acc1 = acc1 + s_rows[eb + j, pl.ds(HALF, HALF)]
            f0 = u_rows[l, pl.ds(0, HALF)] + acc0 * (1.0 / H)
            f1 = u_rows[l, pl.ds(HALF, HALF)] + acc1 * (1.0 / H)
            ps = jnp.sum(f0 * p_rows[l, pl.ds(0, HALF)]
                         + f1 * p_rows[l, pl.ds(HALF, HALF)])
            ns = jnp.sum(f0 * n_rows[l, pl.ds(0, HALF)]
                         + f1 * n_rows[l, pl.ds(HALF, HALF)])
            pos_vec = jnp.where(lane == l, ps, pos_vec)
            neg_vec = jnp.where(lane == l, ns, neg_vec)
            return pos_vec, neg_vec

        z = jnp.zeros((HALF,), jnp.float32)
        pos_vec, neg_vec = lax.fori_loop(0, CB, elem_body, (z, z))
        score_p[pl.ds(c * CB, CB)] = pos_vec
        score_n[pl.ds(c * CB, CB)] = neg_vec

    fire(0, bufs[0])

    def pair_body(cp, _):
        for p in (0, 1):
            c = cp * 2 + p
            cn = lax.rem(c + 1, NCH)
            fire(cn, bufs[1 - p])
            drain(bufs[p])
            compute(c, bufs[p])
        return 0

    lax.fori_loop(0, NCH // 2, pair_body, 0)
    drain(bufs[0])

    pltpu.sync_copy(score_p, out_h.at[0, pl.ds(base_w, BPW)])
    pltpu.sync_copy(score_n, out_h.at[1, pl.ds(base_w, BPW)])


@jax.jit
def _run(users, seqs, posItems, negItems, utv, ut16, emb_item_w):
    emb_user_w = _transpose_user(utv, ut16).reshape(1000000, D)
    mesh = plsc.VectorSubcoreMesh(core_axis_name="c", subcore_axis_name="s",
                                  num_cores=NC, num_subcores=NS)
    f = pl.kernel(
        _sc_body,
        out_type=jax.ShapeDtypeStruct((2, B), jnp.float32),
        mesh=mesh,
        scratch_types=[
            pltpu.VMEM((BPW,), jnp.float32),
            pltpu.VMEM((BPW,), jnp.float32),
            pltpu.VMEM((CB * HP,), jnp.int32),
            pltpu.VMEM((CB * HP,), jnp.int32),
            pltpu.VMEM((CB * H, D), jnp.float32),
            pltpu.VMEM((CB * H, D), jnp.float32),
            pltpu.VMEM((CB,), jnp.int32),
            pltpu.VMEM((CB,), jnp.int32),
            pltpu.VMEM((CB,), jnp.int32),
            pltpu.VMEM((CB,), jnp.int32),
            pltpu.VMEM((CB,), jnp.int32),
            pltpu.VMEM((CB,), jnp.int32),
            pltpu.VMEM((CB, D), jnp.float32),
            pltpu.VMEM((CB, D), jnp.float32),
            pltpu.VMEM((CB, D), jnp.float32),
            pltpu.VMEM((CB, D), jnp.float32),
            pltpu.VMEM((CB, D), jnp.float32),
            pltpu.VMEM((CB, D), jnp.float32),
            pltpu.SemaphoreType.DMA,
            pltpu.SemaphoreType.DMA,
        ],
        compiler_params=pltpu.CompilerParams(use_tc_tiling_on_sc=False,
                                             needs_layout_passes=False),
    )
    return f(users, seqs, posItems, negItems, emb_user_w, emb_item_w)


def kernel(users, seqs, posItems, negItems, emb_user_w, emb_item_w):
    seqs_flat = jnp.pad(seqs, ((0, 0), (0, HP - H))).reshape(-1)
    return _run(users, seqs_flat, posItems, negItems,
                emb_user_w.T,
                emb_user_w[1000000 - TAIL:, :].reshape(TAIL * D // 128, 128),
                emb_item_w)


# speedup vs baseline: 1.1231x; 1.0006x over previous
import jax
import jax.numpy as jnp
from jax import lax
from jax.experimental import pallas as pl
from jax.experimental.pallas import tpu as pltpu
from jax.experimental.pallas import tpu_sc as plsc

B = 16384
H = 50
D = 32
NC, NS = 2, 16
NW = NC * NS
BPW = B // NW
CB = 16
NCH = BPW // CB
HALF = D // 2
HP = 56


UNIT = 512
V_ROWS = 1000000
NUF = V_ROWS // UNIT
TAIL = V_ROWS - NUF * UNIT
KMAX = (NUF + NW - 1) // NW
MPITCH = D + 1


def _tp_body(utv_h, ut16_h, uout_h, win, mid, obuf):
    wid = lax.axis_index("s") * NC + lax.axis_index("c")
    lane = lax.iota(jnp.int32, HALF)

    def do_unit(col0, r0):
        pltpu.sync_copy(utv_h.at[:, pl.ds(col0, UNIT)], win)

        def d_body(d, _):
            for k2 in range(UNIT // HALF):
                vals = win[d, pl.ds(k2 * HALF, HALF)]
                idx = (lane + k2 * HALF) * MPITCH + d
                plsc.store_scatter(mid, [idx], vals)
            return 0

        lax.fori_loop(0, D, d_body, 0)

        def r_body(r, _):
            rb = r * (4 * MPITCH)
            for a in range(4):
                ob = a * D
                obuf[r, pl.ds(ob, HALF)] = mid[pl.ds(rb + a * MPITCH, HALF)]
                obuf[r, pl.ds(ob + HALF, HALF)] = (
                    mid[pl.ds(rb + a * MPITCH + HALF, HALF)])
            return 0

        lax.fori_loop(0, UNIT // 4, r_body, 0)
        pltpu.sync_copy(obuf, uout_h.at[pl.ds(r0, UNIT * D // 128), :])

    def unit_body(k, _):
        u = wid + k * NW

        @pl.when(u < NUF)
        def _():
            do_unit(u * UNIT, u * (UNIT * D // 128))
        return 0

    lax.fori_loop(0, KMAX, unit_body, 0)

    @pl.when(wid == 0)
    def _():
        tr = TAIL * D // 128
        r0 = NUF * (UNIT * D // 128)
        pltpu.sync_copy(ut16_h, obuf.at[pl.ds(0, tr), :])
        pltpu.sync_copy(obuf.at[pl.ds(0, tr), :], uout_h.at[pl.ds(r0, tr), :])


def _transpose_user(utv, ut16):
    mesh = plsc.VectorSubcoreMesh(core_axis_name="c", subcore_axis_name="s",
                                  num_cores=NC, num_subcores=NS)
    f = pl.kernel(
        _tp_body,
        out_type=jax.ShapeDtypeStruct((V_ROWS // 4, 128), jnp.float32),
        mesh=mesh,
        scratch_types=[
            pltpu.VMEM((D, UNIT), jnp.float32),
            pltpu.VMEM((UNIT * MPITCH,), jnp.float32),
            pltpu.VMEM((UNIT * D // 128, 128), jnp.float32),
        ],
        compiler_params=pltpu.CompilerParams(use_tc_tiling_on_sc=True,
                                             needs_layout_passes=False,
                                             disable_bounds_checks=True),
    )
    return f(utv, ut16)


def _sc_body(users_h, seqs_h, pos_h, neg_h, uw_h, iw_h, out_h,
             score_p, score_n,
             s_idx_a, s_idx_b, s_rows_a, s_rows_b,
             u_idx_a, u_idx_b, p_idx_a, p_idx_b, n_idx_a, n_idx_b,
             u_rows_a, u_rows_b, p_rows_a, p_rows_b, n_rows_a, n_rows_b,
             sem_a, sem_b):
    wid = lax.axis_index("s") * NC + lax.axis_index("c")
    base_w = wid * BPW
    lane = lax.iota(jnp.int32, HALF)

    bufs = ((s_idx_a, s_rows_a, u_idx_a, u_rows_a, p_idx_a, p_rows_a,
             n_idx_a, n_rows_a, sem_a),
            (s_idx_b, s_rows_b, u_idx_b, u_rows_b, p_idx_b, p_rows_b,
             n_idx_b, n_rows_b, sem_b))

    def fire(c, buf):
        s_idx, s_rows, u_idx, u_rows, p_idx, p_rows, n_idx, n_rows, sem = buf
        cbase = base_w + c * CB
        pltpu.sync_copy(seqs_h.at[pl.ds(cbase * HP, CB * HP)], s_idx)
        pltpu.sync_copy(users_h.at[pl.ds(cbase, CB)], u_idx)
        pltpu.sync_copy(pos_h.at[pl.ds(cbase, CB)], p_idx)
        pltpu.sync_copy(neg_h.at[pl.ds(cbase, CB)], n_idx)
        pltpu.async_copy(uw_h.at[u_idx], u_rows, sem)
        pltpu.async_copy(iw_h.at[p_idx], p_rows, sem)
        pltpu.async_copy(iw_h.at[n_idx], n_rows, sem)
        for e in range(CB):
            pltpu.async_copy(iw_h.at[s_idx.at[pl.ds(e * HP, H)]],
                             s_rows.at[pl.ds(e * H, H), :], sem)

    def drain(buf):
        s_idx, s_rows, u_idx, u_rows, p_idx, p_rows, n_idx, n_rows, sem = buf
        pltpu.make_async_copy(uw_h.at[u_idx], u_rows, sem).wait()
        pltpu.make_async_copy(iw_h.at[p_idx], p_rows, sem).wait()
        pltpu.make_async_copy(iw_h.at[n_idx], n_rows, sem).wait()
        for e in range(CB):
            pltpu.make_async_copy(iw_h.at[s_idx.at[pl.ds(e * HP, H)]],
                                  s_rows.at[pl.ds(e * H, H), :], sem).wait()

    def compute(c, buf):
        s_idx, s_rows, u_idx, u_rows, p_idx, p_rows, n_idx, n_rows, sem = buf

        def elem_body(l, carry):
            pos_vec, neg_vec = carry
            eb = l * H
            acc0 = s_rows[eb, pl.ds(0, HALF)]
            acc1 = s_rows[eb, pl.ds(HALF, HALF)]
            for j in range(1, H):
                acc0 = acc0 + s_rows[eb + j, pl.ds(0, HALF)]
                acc1 = acc1 + s_rows[eb + j, pl.ds(HALF, HALF)]
            f0 = u_rows[l, pl.ds(0, HALF)] + acc0 * (1.0 / H)
            f1 = u_rows[l, pl.ds(HALF, HALF)] + acc1 * (1.0 / H)
            ps = jnp.sum(f0 * p_rows[l, pl.ds(0, HALF)]
                         + f1 * p_rows[l, pl.ds(HALF, HALF)])
            ns = jnp.sum(f0 * n_rows[l, pl.ds(0, HALF)]
                         + f1 * n_rows[l, pl.ds(HALF, HALF)])
            pos_vec = jnp.where(lane == l, ps, pos_vec)
            neg_vec = jnp.where(lane == l, ns, neg_vec)
            return pos_vec, neg_vec

        z = jnp.zeros((HALF,), jnp.float32)
        pos_vec, neg_vec = lax.fori_loop(0, CB, elem_body, (z, z))
        score_p[pl.ds(c * CB, CB)] = pos_vec
        score_n[pl.ds(c * CB, CB)] = neg_vec

    fire(0, bufs[0])

    def pair_body(cp, _):
        for p in (0, 1):
            c = cp * 2 + p
            cn = lax.rem(c + 1, NCH)
            fire(cn, bufs[1 - p])
            drain(bufs[p])
            compute(c, bufs[p])
        return 0

    lax.fori_loop(0, NCH // 2, pair_body, 0)
    drain(bufs[0])

    pltpu.sync_copy(score_p, out_h.at[0, pl.ds(base_w, BPW)])
    pltpu.sync_copy(score_n, out_h.at[1, pl.ds(base_w, BPW)])


@jax.jit
def _run(users, seqs, posItems, negItems, utv, ut16, emb_item_w):
    emb_user_w = _transpose_user(utv, ut16).reshape(1000000, D)
    mesh = plsc.VectorSubcoreMesh(core_axis_name="c", subcore_axis_name="s",
                                  num_cores=NC, num_subcores=NS)
    f = pl.kernel(
        _sc_body,
        out_type=jax.ShapeDtypeStruct((2, B), jnp.float32),
        mesh=mesh,
        scratch_types=[
            pltpu.VMEM((BPW,), jnp.float32),
            pltpu.VMEM((BPW,), jnp.float32),
            pltpu.VMEM((CB * HP,), jnp.int32),
            pltpu.VMEM((CB * HP,), jnp.int32),
            pltpu.VMEM((CB * H, D), jnp.float32),
            pltpu.VMEM((CB * H, D), jnp.float32),
            pltpu.VMEM((CB,), jnp.int32),
            pltpu.VMEM((CB,), jnp.int32),
            pltpu.VMEM((CB,), jnp.int32),
            pltpu.VMEM((CB,), jnp.int32),
            pltpu.VMEM((CB,), jnp.int32),
            pltpu.VMEM((CB,), jnp.int32),
            pltpu.VMEM((CB, D), jnp.float32),
            pltpu.VMEM((CB, D), jnp.float32),
            pltpu.VMEM((CB, D), jnp.float32),
            pltpu.VMEM((CB, D), jnp.float32),
            pltpu.VMEM((CB, D), jnp.float32),
            pltpu.VMEM((CB, D), jnp.float32),
            pltpu.SemaphoreType.DMA,
            pltpu.SemaphoreType.DMA,
        ],
        compiler_params=pltpu.CompilerParams(use_tc_tiling_on_sc=False,
                                             needs_layout_passes=False),
    )
    return f(users, seqs, posItems, negItems, emb_user_w, emb_item_w)


def kernel(users, seqs, posItems, negItems, emb_user_w, emb_item_w):
    seqs_flat = jnp.pad(seqs, ((0, 0), (0, HP - H))).reshape(-1)
    return _run(users, seqs_flat, posItems, negItems,
                emb_user_w.T,
                emb_user_w[1000000 - TAIL:, :].reshape(TAIL * D // 128, 128),
                emb_item_w)
